# initial kernel scaffold (unmeasured)
import jax
import jax.numpy as jnp
from jax import lax
from jax.experimental import pallas as pl
from jax.experimental.pallas import tpu as pltpu

N_DEV = 8


def kernel(x, w_mat, scale_x, scale_w):
    m_per, k = x.shape
    _, n_per = w_mat.shape

    x8 = x.astype(jnp.float8_e4m3fn)
    w16 = w_mat.astype(jnp.bfloat16)

    def body(x_ref, w_ref, sx_ref, sw_ref, out_ref, xfull, send_sems, recv_sems):
        my = lax.axis_index("i")
        left = (my - 1) % N_DEV
        right = (my + 1) % N_DEV

        barrier_sem = pltpu.get_barrier_semaphore()
        for nbr in (left, right):
            pl.semaphore_signal(
                barrier_sem, inc=1,
                device_id=(nbr,), device_id_type=pl.DeviceIdType.MESH,
            )
        pl.semaphore_wait(barrier_sem, 2)

        scale = sx_ref[0] * sw_ref[0]

        def compute_chunk(origin):
            a = xfull[origin].astype(jnp.bfloat16)
            acc = jnp.dot(a, w_ref[...], preferred_element_type=jnp.float32)
            y = acc * scale
            z = jnp.clip(y, -60.0, 60.0)
            out_ref[pl.ds(origin * m_per, m_per), :] = y / (1.0 + jnp.exp(-z))

        xfull[my] = x_ref[...]
        compute_chunk(my)

        for h in range(N_DEV - 1):
            src_slot = (my - h) % N_DEV
            dst_slot = (my - h - 1) % N_DEV
            rdma = pltpu.make_async_remote_copy(
                src_ref=xfull.at[src_slot],
                dst_ref=xfull.at[src_slot],
                send_sem=send_sems.at[h],
                recv_sem=recv_sems.at[h],
                device_id=(right,),
                device_id_type=pl.DeviceIdType.MESH,
            )
            rdma.start()
            rdma.wait()
            compute_chunk(dst_slot)

    out_shape = jax.ShapeDtypeStruct((N_DEV * m_per, n_per), jnp.float32)
    return pl.pallas_call(
        body,
        out_shape=out_shape,
        in_specs=[
            pl.BlockSpec(memory_space=pltpu.VMEM),
            pl.BlockSpec(memory_space=pltpu.VMEM),
            pl.BlockSpec(memory_space=pltpu.SMEM),
            pl.BlockSpec(memory_space=pltpu.SMEM),
        ],
        out_specs=pl.BlockSpec(memory_space=pltpu.VMEM),
        scratch_shapes=[
            pltpu.VMEM((N_DEV, m_per, k), jnp.float8_e4m3fn),
            pltpu.SemaphoreType.DMA((N_DEV - 1,)),
            pltpu.SemaphoreType.DMA((N_DEV - 1,)),
        ],
        compiler_params=pltpu.CompilerParams(collective_id=0),
    )(x8, w16, scale_x, scale_w)


# baseline (device time: 252155 ns/iter reference)
import jax
import jax.numpy as jnp
from jax import lax
from jax.experimental import pallas as pl
from jax.experimental.pallas import tpu as pltpu

N_DEV = 8


def kernel(x, w_mat, scale_x, scale_w):
    m_per, k = x.shape
    _, n_per = w_mat.shape

    x8 = x.astype(jnp.float8_e4m3fn)
    w16 = w_mat.astype(jnp.bfloat16)

    def body(x_ref, w_ref, sx_ref, sw_ref, out_ref, xfull, send_sems, recv_sems):
        my = lax.axis_index("i")
        left = (my - 1) % N_DEV
        right = (my + 1) % N_DEV

        barrier_sem = pltpu.get_barrier_semaphore()
        for nbr in (left, right):
            pl.semaphore_signal(
                barrier_sem, inc=1,
                device_id=(nbr,), device_id_type=pl.DeviceIdType.MESH,
            )
        pl.semaphore_wait(barrier_sem, 2)

        scale = sx_ref[0] * sw_ref[0]

        def compute_chunk(origin):
            a = xfull[origin].astype(jnp.bfloat16)
            acc = jnp.dot(a, w_ref[...], preferred_element_type=jnp.float32)
            y = acc * scale
            z = jnp.clip(y, -60.0, 60.0)
            out_ref[pl.ds(origin * m_per, m_per), :] = y / (1.0 + jnp.exp(-z))

        xfull[my] = x_ref[...]
        compute_chunk(my)

        for h in range(N_DEV - 1):
            src_slot = (my - h) % N_DEV
            dst_slot = (my - h - 1) % N_DEV
            rdma = pltpu.make_async_remote_copy(
                src_ref=xfull.at[src_slot],
                dst_ref=xfull.at[src_slot],
                send_sem=send_sems.at[h],
                recv_sem=recv_sems.at[h],
                device_id=(right,),
                device_id_type=pl.DeviceIdType.MESH,
            )
            rdma.start()
            rdma.wait()
            compute_chunk(dst_slot)

    out_shape = jax.ShapeDtypeStruct((N_DEV * m_per, n_per), jnp.float32)
    return pl.pallas_call(
        body,
        out_shape=out_shape,
        in_specs=[
            pl.BlockSpec(memory_space=pltpu.VMEM),
            pl.BlockSpec(memory_space=pltpu.VMEM),
            pl.BlockSpec(memory_space=pltpu.SMEM),
            pl.BlockSpec(memory_space=pltpu.SMEM),
        ],
        out_specs=pl.BlockSpec(memory_space=pltpu.VMEM),
        scratch_shapes=[
            pltpu.VMEM((N_DEV, m_per, k), jnp.float8_e4m3fn),
            pltpu.SemaphoreType.DMA((N_DEV - 1,)),
            pltpu.SemaphoreType.DMA((N_DEV - 1,)),
        ],
        compiler_params=pltpu.CompilerParams(
            collective_id=0, vmem_limit_bytes=100 * 1024 * 1024
        ),
    )(x8, w16, scale_x, scale_w)


# device time: 123700 ns/iter; 2.0384x vs baseline; 2.0384x over previous
import jax
import jax.numpy as jnp
from jax import lax
from jax.experimental import pallas as pl
from jax.experimental.pallas import tpu as pltpu

N_DEV = 8


def kernel(x, w_mat, scale_x, scale_w):
    m_per, k = x.shape
    _, n_per = w_mat.shape

    x8 = x.astype(jnp.float8_e4m3fn)
    w16 = w_mat.astype(jnp.bfloat16)

    def body(x_ref, w_ref, sx_ref, sw_ref, out_ref, xfull, send_sems, recv_sems):
        my = lax.axis_index("i")
        r = my % 4
        b = my - r
        o = 4 - b
        cw = b + (r + 1) % 4
        ccw = b + (r + 3) % 4
        zp = o + r

        s_ccw1 = b + (r + 3) % 4
        s_ccw2 = b + (r + 2) % 4
        s_cw1 = b + (r + 1) % 4
        o_own = o + r
        o_ccw1 = o + (r + 3) % 4
        o_cw1 = o + (r + 1) % 4
        o_cw2 = o + (r + 2) % 4

        barrier_sem = pltpu.get_barrier_semaphore()
        for nbr in (cw, ccw, zp):
            pl.semaphore_signal(
                barrier_sem, inc=1,
                device_id=(nbr,), device_id_type=pl.DeviceIdType.MESH,
            )
        pl.semaphore_wait(barrier_sem, 3)

        def copy(tid, slot, dst):
            return pltpu.make_async_remote_copy(
                src_ref=xfull.at[slot],
                dst_ref=xfull.at[slot],
                send_sem=send_sems.at[tid],
                recv_sem=recv_sems.at[tid],
                device_id=(dst,),
                device_id_type=pl.DeviceIdType.MESH,
            )

        def wait_inbound(tid, slot):
            copy(tid, slot, cw).wait_recv()

        scale = sx_ref[0] * sw_ref[0]

        def compute_chunk(slot):
            a = xfull[slot].astype(jnp.bfloat16)
            acc = jnp.dot(a, w_ref[...], preferred_element_type=jnp.float32)
            y = acc * scale
            z = jnp.clip(y, -60.0, 60.0)
            out_ref[pl.ds(slot * m_per, m_per), :] = y / (1.0 + jnp.exp(-z))

        xfull[my] = x_ref[...]
        t0 = copy(0, my, cw)
        t3 = copy(3, my, ccw)
        t5 = copy(5, my, zp)
        t0.start()
        t3.start()
        t5.start()

        compute_chunk(my)

        wait_inbound(0, s_ccw1)
        t1 = copy(1, s_ccw1, cw)
        t1.start()
        wait_inbound(5, o_own)
        t2 = copy(2, o_own, cw)
        t4 = copy(4, o_own, ccw)
        t2.start()
        t4.start()

        compute_chunk(s_ccw1)
        compute_chunk(o_own)

        wait_inbound(1, s_ccw2)
        t6 = copy(6, s_ccw2, zp)
        t6.start()
        compute_chunk(s_ccw2)

        wait_inbound(3, s_cw1)
        compute_chunk(s_cw1)
        wait_inbound(4, o_cw1)
        compute_chunk(o_cw1)
        wait_inbound(2, o_ccw1)
        compute_chunk(o_ccw1)
        wait_inbound(6, o_cw2)
        compute_chunk(o_cw2)

        for t in (t0, t1, t2, t3, t4, t5, t6):
            t.wait_send()

    out_shape = jax.ShapeDtypeStruct((N_DEV * m_per, n_per), jnp.float32)
    return pl.pallas_call(
        body,
        out_shape=out_shape,
        in_specs=[
            pl.BlockSpec(memory_space=pltpu.VMEM),
            pl.BlockSpec(memory_space=pltpu.VMEM),
            pl.BlockSpec(memory_space=pltpu.SMEM),
            pl.BlockSpec(memory_space=pltpu.SMEM),
        ],
        out_specs=pl.BlockSpec(memory_space=pltpu.VMEM),
        scratch_shapes=[
            pltpu.VMEM((N_DEV, m_per, k), jnp.float8_e4m3fn),
            pltpu.SemaphoreType.DMA((7,)),
            pltpu.SemaphoreType.DMA((7,)),
        ],
        compiler_params=pltpu.CompilerParams(
            collective_id=0, vmem_limit_bytes=100 * 1024 * 1024
        ),
    )(x8, w16, scale_x, scale_w)


# device time: 86915 ns/iter; 2.9012x vs baseline; 1.4232x over previous
import jax
import jax.numpy as jnp
from jax import lax
from jax.experimental import pallas as pl
from jax.experimental.pallas import tpu as pltpu

N_DEV = 8


def kernel(x, w_mat, scale_x, scale_w):
    m_per, k = x.shape
    _, n_per = w_mat.shape
    mh = m_per // 2

    def body(x_ref, w_ref, sx_ref, sw_ref, out_ref, xfull, w8, wtmp,
             otmp, wdma_sem, odma_sem, send_sems, recv_sems):
        my = lax.axis_index("i")
        r = my % 4
        b = my - r
        o = 4 - b
        cw = b + (r + 1) % 4
        ccw = b + (r + 3) % 4
        zp = o + r

        s_ccw1 = b + (r + 3) % 4
        s_ccw2 = b + (r + 2) % 4
        s_cw1 = b + (r + 1) % 4
        o_own = o + r
        o_ccw1 = o + (r + 3) % 4
        o_cw1 = o + (r + 1) % 4
        o_cw2 = o + (r + 2) % 4

        barrier_sem = pltpu.get_barrier_semaphore()
        for nbr in (cw, ccw, zp):
            pl.semaphore_signal(
                barrier_sem, inc=1,
                device_id=(nbr,), device_id_type=pl.DeviceIdType.MESH,
            )
        pl.semaphore_wait(barrier_sem, 3)

        def copy(tid, slot, half, dst):
            ref = xfull.at[slot, pl.ds(half * mh, mh), :]
            return pltpu.make_async_remote_copy(
                src_ref=ref,
                dst_ref=ref,
                send_sem=send_sems.at[tid],
                recv_sem=recv_sems.at[tid],
                device_id=(dst,),
                device_id_type=pl.DeviceIdType.MESH,
            )

        def wait_inbound(tid, slot, half):
            copy(tid, slot, half, cw).wait_recv()

        scale = sx_ref[0] * sw_ref[0]

        ocopies = []

        def compute_chunk(slot):
            acc = lax.dot_general(
                xfull[slot], w8[...],
                (((1,), (0,)), ((), ())),
                preferred_element_type=jnp.float32,
            )
            y = acc * scale
            z = jnp.clip(y, -60.0, 60.0)
            par = len(ocopies) % 2
            if len(ocopies) >= 2:
                ocopies[-2].wait()
            otmp[par] = y / (1.0 + jnp.exp(-z))
            c = pltpu.make_async_copy(
                otmp.at[par],
                out_ref.at[pl.ds(slot * m_per, m_per), :],
                odma_sem.at[par],
            )
            c.start()
            ocopies.append(c)

        xfull[my, pl.ds(0, mh), :] = x_ref[pl.ds(0, mh), :].astype(
            jnp.float8_e4m3fn)
        t0 = copy(0, my, 0, cw)
        t5 = copy(5, my, 0, ccw)
        t9 = copy(9, my, 0, zp)
        for t in (t0, t5, t9):
            t.start()
        xfull[my, pl.ds(mh, mh), :] = x_ref[pl.ds(mh, mh), :].astype(
            jnp.float8_e4m3fn)
        t1 = copy(1, my, 1, cw)
        t6 = copy(6, my, 1, ccw)
        t10 = copy(10, my, 1, zp)
        for t in (t1, t6, t10):
            t.start()

        kq = k // 4
        wcopies = [
            pltpu.make_async_copy(
                w_ref.at[pl.ds(q * kq, kq), :],
                wtmp.at[q % 2],
                wdma_sem.at[q % 2],
            )
            for q in range(4)
        ]
        wcopies[0].start()
        wcopies[1].start()
        for q in range(4):
            wcopies[q].wait()
            w8[pl.ds(q * kq, kq), :] = wtmp[q % 2].astype(jnp.float8_e5m2)
            if q + 2 < 4:
                wcopies[q + 2].start()

        compute_chunk(my)

        wait_inbound(0, s_ccw1, 0)
        t2 = copy(2, s_ccw1, 0, cw)
        t2.start()
        wait_inbound(9, o_own, 0)
        t3 = copy(3, o_own, 0, cw)
        t8 = copy(8, o_own, 0, ccw)
        t3.start()
        t8.start()
        wait_inbound(6, s_cw1, 1)
        t7 = copy(7, s_cw1, 1, ccw)
        t11 = copy(11, s_cw1, 1, zp)
        t7.start()
        t11.start()
        wait_inbound(10, o_own, 1)
        t4 = copy(4, o_own, 1, cw)
        t4.start()
        compute_chunk(o_own)

        wait_inbound(2, s_ccw2, 0)
        t12 = copy(12, s_ccw2, 0, zp)
        t12.start()
        wait_inbound(11, o_cw1, 1)
        t13 = copy(13, o_cw1, 1, ccw)
        t13.start()
        wait_inbound(7, s_ccw2, 1)
        compute_chunk(s_ccw2)

        wait_inbound(1, s_ccw1, 1)
        compute_chunk(s_ccw1)
        wait_inbound(5, s_cw1, 0)
        compute_chunk(s_cw1)
        wait_inbound(8, o_cw1, 0)
        compute_chunk(o_cw1)
        def compute_half(slot, half, par):
            rows = pl.ds(half * mh, mh)
            acc = lax.dot_general(
                xfull[slot, rows, :], w8[...],
                (((1,), (0,)), ((), ())),
                preferred_element_type=jnp.float32,
            )
            y = acc * scale
            z = jnp.clip(y, -60.0, 60.0)
            otmp[par, rows, :] = y / (1.0 + jnp.exp(-z))

        wait_inbound(3, o_ccw1, 0)
        ocopies[-2].wait()
        compute_half(o_ccw1, 0, 0)
        wait_inbound(12, o_cw2, 0)
        ocopies[-1].wait()
        compute_half(o_cw2, 0, 1)
        wait_inbound(4, o_ccw1, 1)
        compute_half(o_ccw1, 1, 0)
        c0 = pltpu.make_async_copy(
            otmp.at[0],
            out_ref.at[pl.ds(o_ccw1 * m_per, m_per), :],
            odma_sem.at[0],
        )
        c0.start()
        wait_inbound(13, o_cw2, 1)
        compute_half(o_cw2, 1, 1)
        c1 = pltpu.make_async_copy(
            otmp.at[1],
            out_ref.at[pl.ds(o_cw2 * m_per, m_per), :],
            odma_sem.at[1],
        )
        c1.start()

        c0.wait()
        c1.wait()
        for t in (t0, t1, t2, t3, t4, t5, t6, t7, t8, t9, t10, t11, t12, t13):
            t.wait_send()

    out_shape = jax.ShapeDtypeStruct((N_DEV * m_per, n_per), jnp.float32)
    return pl.pallas_call(
        body,
        out_shape=out_shape,
        in_specs=[
            pl.BlockSpec(memory_space=pltpu.VMEM),
            pl.BlockSpec(memory_space=pl.ANY),
            pl.BlockSpec(memory_space=pltpu.SMEM),
            pl.BlockSpec(memory_space=pltpu.SMEM),
        ],
        out_specs=pl.BlockSpec(memory_space=pl.ANY),
        scratch_shapes=[
            pltpu.VMEM((N_DEV, m_per, k), jnp.float8_e4m3fn),
            pltpu.VMEM((k, n_per), jnp.float8_e5m2),
            pltpu.VMEM((2, k // 4, n_per), jnp.float32),
            pltpu.VMEM((2, m_per, n_per), jnp.float32),
            pltpu.SemaphoreType.DMA((2,)),
            pltpu.SemaphoreType.DMA((2,)),
            pltpu.SemaphoreType.DMA((14,)),
            pltpu.SemaphoreType.DMA((14,)),
        ],
        compiler_params=pltpu.CompilerParams(
            collective_id=0, vmem_limit_bytes=100 * 1024 * 1024
        ),
    )(x, w_mat, scale_x, scale_w)
